# baseline (speedup 1.0000x reference)
import functools

import jax
import jax.numpy as jnp
from jax.experimental import pallas as pl
from jax.experimental.pallas import tpu as pltpu

N = 32768
B = 16
D = 1024
EPS = 1e-05

ROWS = 512


def _onehot(off_ref, step, rows):
    r = step * rows + jax.lax.broadcasted_iota(jnp.int32, (rows, 1), 0)
    cols = []
    for b in range(B):
        start = off_ref[b - 1] if b > 0 else 0
        end = off_ref[b]
        cols.append(((r >= start) & (r < end)).astype(jnp.float32))
    return jnp.concatenate(cols, axis=1)


def _stats_kernel(off_ref, x_ref, sum_ref, sq_ref):
    step = pl.program_id(0)
    oh = _onehot(off_ref, step, ROWS)
    x = x_ref[...]
    dims = (((0,), (0,)), ((), ()))
    ps = jax.lax.dot_general(oh, x, dims, preferred_element_type=jnp.float32)
    psq = jax.lax.dot_general(oh, x * x, dims,
                              preferred_element_type=jnp.float32)

    @pl.when(step == 0)
    def _():
        sum_ref[...] = ps
        sq_ref[...] = psq

    @pl.when(step != 0)
    def _():
        sum_ref[...] += ps
        sq_ref[...] += psq


def _norm_kernel(off_ref, x_ref, sum_ref, sq_ref, w_ref, b_ref, out_ref):
    step = pl.program_id(0)
    lens = []
    for b in range(B):
        start = off_ref[b - 1] if b > 0 else 0
        lens.append(jnp.maximum(off_ref[b] - start, 1))
    inv_cnt = 1.0 / jnp.stack(lens).astype(jnp.float32).reshape(B, 1)
    s = sum_ref[...]
    sq = sq_ref[...]
    mean = s * inv_cnt
    var = sq * inv_cnt - mean * mean
    rstd = jax.lax.rsqrt(jnp.maximum(var, 0.0) + EPS)
    scale = rstd * w_ref[...]
    shift = b_ref[...] - mean * scale
    oh = _onehot(off_ref, step, ROWS)
    dims = (((1,), (0,)), ((), ()))
    row_scale = jax.lax.dot_general(oh, scale, dims,
                                    preferred_element_type=jnp.float32)
    row_shift = jax.lax.dot_general(oh, shift, dims,
                                    preferred_element_type=jnp.float32)
    out_ref[...] = x_ref[...] * row_scale + row_shift


@functools.partial(jax.jit, static_argnames=("interpret",))
def kernel(input, offsets, weight, bias, interpret=False):
    steps = N // ROWS
    stats_grid = pltpu.PrefetchScalarGridSpec(
        num_scalar_prefetch=1,
        grid=(steps,),
        in_specs=[pl.BlockSpec((ROWS, D), lambda i, off: (i, 0))],
        out_specs=[pl.BlockSpec((B, D), lambda i, off: (0, 0)),
                   pl.BlockSpec((B, D), lambda i, off: (0, 0))],
    )
    ssum, ssq = pl.pallas_call(
        _stats_kernel,
        grid_spec=stats_grid,
        out_shape=[jax.ShapeDtypeStruct((B, D), jnp.float32),
                   jax.ShapeDtypeStruct((B, D), jnp.float32)],
        interpret=interpret,
    )(offsets, input)

    norm_grid = pltpu.PrefetchScalarGridSpec(
        num_scalar_prefetch=1,
        grid=(steps,),
        in_specs=[pl.BlockSpec((ROWS, D), lambda i, off: (i, 0)),
                  pl.BlockSpec((B, D), lambda i, off: (0, 0)),
                  pl.BlockSpec((B, D), lambda i, off: (0, 0)),
                  pl.BlockSpec((1, D), lambda i, off: (0, 0)),
                  pl.BlockSpec((1, D), lambda i, off: (0, 0))],
        out_specs=pl.BlockSpec((ROWS, D), lambda i, off: (i, 0)),
    )
    out = pl.pallas_call(
        _norm_kernel,
        grid_spec=norm_grid,
        out_shape=jax.ShapeDtypeStruct((N, D), jnp.float32),
        interpret=interpret,
    )(offsets, input, ssum, ssq,
      weight.reshape(1, D), bias.reshape(1, D))
    return out


# speedup vs baseline: 1.2943x; 1.2943x over previous
import functools

import jax
import jax.numpy as jnp
from jax.experimental import pallas as pl
from jax.experimental.pallas import tpu as pltpu

N = 32768
B = 16
D = 1024
EPS = 1e-05

ROWS = 2048


def _onehot(off_ref, step, rows):
    r = step * rows + jax.lax.broadcasted_iota(jnp.int32, (rows, 1), 0)
    cols = []
    for b in range(B):
        start = off_ref[b - 1] if b > 0 else 0
        end = off_ref[b]
        cols.append(((r >= start) & (r < end)).astype(jnp.float32))
    return jnp.concatenate(cols, axis=1)


def _stats_kernel(off_ref, x_ref, sum_ref, sq_ref):
    step = pl.program_id(0)
    oh = _onehot(off_ref, step, ROWS)
    x = x_ref[...]
    dims = (((0,), (0,)), ((), ()))
    ps = jax.lax.dot_general(oh, x, dims, preferred_element_type=jnp.float32)
    psq = jax.lax.dot_general(oh, x * x, dims,
                              preferred_element_type=jnp.float32)

    @pl.when(step == 0)
    def _():
        sum_ref[...] = ps
        sq_ref[...] = psq

    @pl.when(step != 0)
    def _():
        sum_ref[...] += ps
        sq_ref[...] += psq


def _norm_kernel(off_ref, x_ref, sum_ref, sq_ref, w_ref, b_ref, out_ref):
    step = pl.program_id(0)
    lens = []
    for b in range(B):
        start = off_ref[b - 1] if b > 0 else 0
        lens.append(jnp.maximum(off_ref[b] - start, 1))
    inv_cnt = 1.0 / jnp.stack(lens).astype(jnp.float32).reshape(B, 1)
    s = sum_ref[...]
    sq = sq_ref[...]
    mean = s * inv_cnt
    var = sq * inv_cnt - mean * mean
    rstd = jax.lax.rsqrt(jnp.maximum(var, 0.0) + EPS)
    scale = rstd * w_ref[...]
    shift = b_ref[...] - mean * scale
    oh = _onehot(off_ref, step, ROWS)
    dims = (((1,), (0,)), ((), ()))
    row_scale = jax.lax.dot_general(oh, scale, dims,
                                    preferred_element_type=jnp.float32)
    row_shift = jax.lax.dot_general(oh, shift, dims,
                                    preferred_element_type=jnp.float32)
    out_ref[...] = x_ref[...] * row_scale + row_shift


@functools.partial(jax.jit, static_argnames=("interpret",))
def kernel(input, offsets, weight, bias, interpret=False):
    steps = N // ROWS
    stats_grid = pltpu.PrefetchScalarGridSpec(
        num_scalar_prefetch=1,
        grid=(steps,),
        in_specs=[pl.BlockSpec((ROWS, D), lambda i, off: (i, 0))],
        out_specs=[pl.BlockSpec((B, D), lambda i, off: (0, 0)),
                   pl.BlockSpec((B, D), lambda i, off: (0, 0))],
    )
    ssum, ssq = pl.pallas_call(
        _stats_kernel,
        grid_spec=stats_grid,
        out_shape=[jax.ShapeDtypeStruct((B, D), jnp.float32),
                   jax.ShapeDtypeStruct((B, D), jnp.float32)],
        interpret=interpret,
    )(offsets, input)

    norm_grid = pltpu.PrefetchScalarGridSpec(
        num_scalar_prefetch=1,
        grid=(steps,),
        in_specs=[pl.BlockSpec((ROWS, D), lambda i, off: (i, 0)),
                  pl.BlockSpec((B, D), lambda i, off: (0, 0)),
                  pl.BlockSpec((B, D), lambda i, off: (0, 0)),
                  pl.BlockSpec((1, D), lambda i, off: (0, 0)),
                  pl.BlockSpec((1, D), lambda i, off: (0, 0))],
        out_specs=pl.BlockSpec((ROWS, D), lambda i, off: (i, 0)),
    )
    out = pl.pallas_call(
        _norm_kernel,
        grid_spec=norm_grid,
        out_shape=jax.ShapeDtypeStruct((N, D), jnp.float32),
        interpret=interpret,
    )(offsets, input, ssum, ssq,
      weight.reshape(1, D), bias.reshape(1, D))
    return out


# speedup vs baseline: 1.6279x; 1.2577x over previous
import functools

import jax
import jax.numpy as jnp
from jax.experimental import pallas as pl

N = 32768
B = 16
D = 1024
EPS = 1e-05

ROWS = 2048


def _onehot(starts_ref, ends_ref, step):
    r = step * ROWS + jax.lax.broadcasted_iota(jnp.int32, (ROWS, B), 0)
    return ((r >= starts_ref[...]) & (r < ends_ref[...])).astype(jnp.float32)


def _stats_kernel(x_ref, starts_ref, ends_ref, sum_ref, sq_ref):
    step = pl.program_id(0)
    oh = _onehot(starts_ref, ends_ref, step)
    x = x_ref[...]
    dims = (((0,), (0,)), ((), ()))
    ps = jax.lax.dot_general(oh, x, dims, preferred_element_type=jnp.float32)
    psq = jax.lax.dot_general(oh, x * x, dims,
                              preferred_element_type=jnp.float32)

    @pl.when(step == 0)
    def _():
        sum_ref[...] = ps
        sq_ref[...] = psq

    @pl.when(step != 0)
    def _():
        sum_ref[...] += ps
        sq_ref[...] += psq


def _prep_kernel(sum_ref, sq_ref, w_ref, b_ref, invc_ref,
                 scale_ref, shift_ref):
    inv = invc_ref[:, 0:1]
    mean = sum_ref[...] * inv
    var = sq_ref[...] * inv - mean * mean
    rstd = jax.lax.rsqrt(jnp.maximum(var, 0.0) + EPS)
    scale = rstd * w_ref[...]
    scale_ref[...] = scale
    shift_ref[...] = b_ref[...] - mean * scale


def _norm_kernel(x_ref, scale_ref, shift_ref, starts_ref, ends_ref, out_ref):
    step = pl.program_id(0)
    oh = _onehot(starts_ref, ends_ref, step)
    dims = (((1,), (0,)), ((), ()))
    row_scale = jax.lax.dot_general(oh, scale_ref[...], dims,
                                    preferred_element_type=jnp.float32)
    row_shift = jax.lax.dot_general(oh, shift_ref[...], dims,
                                    preferred_element_type=jnp.float32)
    out_ref[...] = x_ref[...] * row_scale + row_shift


@functools.partial(jax.jit, static_argnames=("interpret",))
def kernel(input, offsets, weight, bias, interpret=False):
    steps = N // ROWS
    ends = offsets.reshape(1, B)
    starts = jnp.concatenate(
        [jnp.zeros((1, 1), jnp.int32), ends[:, :-1]], axis=1)
    invc = jnp.broadcast_to(
        (1.0 / jnp.maximum(ends - starts, 1).astype(jnp.float32)).reshape(
            B, 1), (B, 128))

    small = pl.BlockSpec((1, B), lambda i: (0, 0))
    ssum, ssq = pl.pallas_call(
        _stats_kernel,
        grid=(steps,),
        in_specs=[pl.BlockSpec((ROWS, D), lambda i: (i, 0)), small, small],
        out_specs=[pl.BlockSpec((B, D), lambda i: (0, 0)),
                   pl.BlockSpec((B, D), lambda i: (0, 0))],
        out_shape=[jax.ShapeDtypeStruct((B, D), jnp.float32),
                   jax.ShapeDtypeStruct((B, D), jnp.float32)],
        interpret=interpret,
    )(input, starts, ends)

    scale, shift = pl.pallas_call(
        _prep_kernel,
        out_shape=[jax.ShapeDtypeStruct((B, D), jnp.float32),
                   jax.ShapeDtypeStruct((B, D), jnp.float32)],
        interpret=interpret,
    )(ssum, ssq, weight.reshape(1, D), bias.reshape(1, D), invc)

    out = pl.pallas_call(
        _norm_kernel,
        grid=(steps,),
        in_specs=[pl.BlockSpec((ROWS, D), lambda i: (i, 0)),
                  pl.BlockSpec((B, D), lambda i: (0, 0)),
                  pl.BlockSpec((B, D), lambda i: (0, 0)),
                  small, small],
        out_specs=pl.BlockSpec((ROWS, D), lambda i: (i, 0)),
        out_shape=jax.ShapeDtypeStruct((N, D), jnp.float32),
        interpret=interpret,
    )(input, scale, shift, starts, ends)
    return out
